# baseline (speedup 1.0000x reference)
import functools

import jax
import jax.numpy as jnp
from jax import lax
from jax.experimental import pallas as pl
from jax.experimental.pallas import tpu as pltpu
from jax.experimental.pallas import tpu_sc as plsc


@functools.lru_cache(maxsize=None)
def _make_gather(V2, D2, B):
    info = plsc.get_sparse_core_info()
    NC, NS = info.num_cores, info.num_subcores
    NW = NC * NS
    assert B % (8 * NW) == 0
    b_per_w = B // NW
    mesh = plsc.VectorSubcoreMesh(core_axis_name="c", subcore_axis_name="s")

    @functools.partial(
        pl.kernel,
        mesh=mesh,
        out_type=jax.ShapeDtypeStruct((B, D2), jnp.float32),
        scratch_types=[
            pltpu.VMEM((b_per_w,), jnp.int32),
            pltpu.VMEM((b_per_w, D2), jnp.float32),
            pltpu.SemaphoreType.DMA,
        ],
    )
    def gather(table2_hbm, ids_hbm, out_hbm, idx_v, out_v, sem):
        wid = lax.axis_index("s") * NC + lax.axis_index("c")
        base = wid * b_per_w
        pltpu.sync_copy(ids_hbm.at[pl.ds(base, b_per_w)], idx_v)

        def fire(g, carry):
            vec = idx_v[pl.ds(g * 16, 16)]
            for l in range(16):
                p = lax.shift_right_logical(vec[l], 1)
                pltpu.make_async_copy(
                    table2_hbm.at[p], out_v.at[g * 16 + l], sem
                ).start()
            return carry

        lax.fori_loop(0, b_per_w // 16, fire, 0)
        pltpu.make_async_copy(
            table2_hbm.at[pl.ds(0, b_per_w)], out_v, sem
        ).wait()
        pltpu.sync_copy(out_v, out_hbm.at[pl.ds(base, b_per_w)])

    return gather


def _mm_body(scale_ref, ids_ref, h2_ref, w2_ref, o_ref):
    h2 = h2_ref[...]
    blk, d2 = h2.shape
    par = ids_ref[...] & 1
    half = lax.broadcasted_iota(jnp.int32, (blk, d2), 1) // (d2 // 2)
    hm = jnp.where(half == par, h2, 0.0)
    acc = lax.dot_general(
        hm,
        w2_ref[...],
        (((1,), (1,)), ((), ())),
        preferred_element_type=jnp.float32,
    )
    o_ref[...] = acc * scale_ref[0]


@functools.lru_cache(maxsize=None)
def _make_matmul(B, D2, MD, blk):
    return pl.pallas_call(
        _mm_body,
        grid=(B // blk,),
        in_specs=[
            pl.BlockSpec(memory_space=pltpu.SMEM),
            pl.BlockSpec((blk, 1), lambda i: (i, 0)),
            pl.BlockSpec((blk, D2), lambda i: (i, 0)),
            pl.BlockSpec((MD, D2), lambda i: (0, 0)),
        ],
        out_specs=pl.BlockSpec((blk, MD), lambda i: (i, 0)),
        out_shape=jax.ShapeDtypeStruct((B, MD), jnp.float32),
    )


def kernel(ids, embed_weight, proj_weight, scale):
    B = ids.shape[0]
    V, D = embed_weight.shape
    MD = proj_weight.shape[0]
    ids = ids.astype(jnp.int32)
    table2 = embed_weight.reshape(V // 2, 2 * D)
    h2 = _make_gather(V // 2, 2 * D, B)(table2, ids)
    w2 = jnp.concatenate([proj_weight, proj_weight], axis=1)
    mm = _make_matmul(B, 2 * D, MD, 512)
    return mm(scale.reshape(1).astype(jnp.float32), ids.reshape(B, 1), h2, w2)


# speedup vs baseline: 2.4817x; 2.4817x over previous
import functools

import jax
import jax.numpy as jnp
from jax import lax
from jax.experimental import pallas as pl
from jax.experimental.pallas import tpu as pltpu
from jax.experimental.pallas import tpu_sc as plsc

_CHUNK = 64


@functools.lru_cache(maxsize=None)
def _make_gather(V, D, B):
    info = plsc.get_sparse_core_info()
    NC, NS = info.num_cores, info.num_subcores
    NW = NC * NS
    assert B % (8 * NW) == 0 and V % 8 == 0
    b_per_w = B // NW
    ch = min(_CHUNK, b_per_w)
    n_ch = b_per_w // ch
    assert b_per_w % ch == 0
    mesh = plsc.VectorSubcoreMesh(core_axis_name="c", subcore_axis_name="s")

    @functools.partial(
        pl.kernel,
        mesh=mesh,
        out_type=jax.ShapeDtypeStruct((B, D), jnp.float32),
        scratch_types=[
            pltpu.VMEM((b_per_w,), jnp.int32),
            pltpu.VMEM((b_per_w, D), jnp.float32),
            pltpu.SemaphoreType.DMA,
        ],
    )
    def gather(table3_hbm, ids_hbm, out_hbm, idx_v, out_v, sem):
        wid = lax.axis_index("s") * NC + lax.axis_index("c")
        base = wid * b_per_w
        pltpu.sync_copy(ids_hbm.at[pl.ds(base, b_per_w)], idx_v)

        def fire(g, carry):
            vec = idx_v[pl.ds(g * 16, 16)]
            for l in range(16):
                sid = vec[l]
                t = lax.shift_right_logical(sid, 3)
                s = sid & 7
                pltpu.make_async_copy(
                    table3_hbm.at[t, s], out_v.at[g * 16 + l], sem
                ).start()
            return carry

        lax.fori_loop(0, b_per_w // 16, fire, 0)
        pltpu.make_async_copy(
            table3_hbm.reshape(V, D).at[pl.ds(0, b_per_w)], out_v, sem
        ).wait()
        pltpu.sync_copy(out_v, out_hbm.at[pl.ds(base, b_per_w)])

    return gather


def _mm_body(scale_ref, h_ref, w_ref, o_ref):
    acc = lax.dot_general(
        h_ref[...],
        w_ref[...],
        (((1,), (1,)), ((), ())),
        preferred_element_type=jnp.float32,
    )
    o_ref[...] = acc * scale_ref[0]


@functools.lru_cache(maxsize=None)
def _make_matmul(B, D, MD, blk):
    return pl.pallas_call(
        _mm_body,
        grid=(B // blk,),
        in_specs=[
            pl.BlockSpec(memory_space=pltpu.SMEM),
            pl.BlockSpec((blk, D), lambda i: (i, 0)),
            pl.BlockSpec((MD, D), lambda i: (0, 0)),
        ],
        out_specs=pl.BlockSpec((blk, MD), lambda i: (i, 0)),
        out_shape=jax.ShapeDtypeStruct((B, MD), jnp.float32),
    )


def kernel(ids, embed_weight, proj_weight, scale):
    B = ids.shape[0]
    V, D = embed_weight.shape
    MD = proj_weight.shape[0]
    ids = ids.astype(jnp.int32)
    table3 = embed_weight.reshape(V // 8, 8, D)
    h = _make_gather(V, D, B)(table3, ids)
    mm = _make_matmul(B, D, MD, 1024)
    return mm(scale.reshape(1).astype(jnp.float32), h, proj_weight)


# speedup vs baseline: 2.5182x; 1.0147x over previous
import functools

import jax
import jax.numpy as jnp
from jax import lax
from jax.experimental import pallas as pl
from jax.experimental.pallas import tpu as pltpu
from jax.experimental.pallas import tpu_sc as plsc

_CHUNK = 64


@functools.lru_cache(maxsize=None)
def _make_gather(V, D, B):
    info = plsc.get_sparse_core_info()
    NC, NS = info.num_cores, info.num_subcores
    NW = NC * NS
    assert B % (8 * NW) == 0 and V % 8 == 0
    b_per_w = B // NW
    ch = min(_CHUNK, b_per_w)
    n_ch = b_per_w // ch
    assert b_per_w % ch == 0
    mesh = plsc.VectorSubcoreMesh(core_axis_name="c", subcore_axis_name="s")

    @functools.partial(
        pl.kernel,
        mesh=mesh,
        out_type=jax.ShapeDtypeStruct((B, D), jnp.float32),
        scratch_types=[
            pltpu.VMEM((b_per_w,), jnp.int32),
            pltpu.VMEM((b_per_w, D), jnp.float32),
            pltpu.SemaphoreType.DMA,
        ],
    )
    def gather(table3_hbm, ids_hbm, out_hbm, idx_v, out_v, sem):
        wid = lax.axis_index("s") * NC + lax.axis_index("c")
        base = wid * b_per_w
        pltpu.sync_copy(ids_hbm.at[pl.ds(base, b_per_w)], idx_v)

        def fire(g, carry):
            vec = idx_v[pl.ds(g * 16, 16)]
            for l in range(16):
                sid = vec[l]
                t = lax.shift_right_logical(sid, 3)
                s = sid & 7
                pltpu.make_async_copy(
                    table3_hbm.at[t, s], out_v.at[g * 16 + l], sem
                ).start()
            return carry

        lax.fori_loop(0, b_per_w // 16, fire, 0)
        pltpu.make_async_copy(
            table3_hbm.reshape(V, D).at[pl.ds(0, b_per_w)], out_v, sem
        ).wait()
        pltpu.sync_copy(out_v, out_hbm.at[pl.ds(base, b_per_w)])

    return gather


def _mm_body(scale_ref, h_ref, w_ref, o_ref):
    acc = lax.dot_general(
        h_ref[...],
        w_ref[...],
        (((1,), (1,)), ((), ())),
        preferred_element_type=jnp.float32,
    )
    o_ref[...] = acc * scale_ref[0]


@functools.lru_cache(maxsize=None)
def _make_matmul(B, D, MD, blk):
    return pl.pallas_call(
        _mm_body,
        grid=(B // blk,),
        in_specs=[
            pl.BlockSpec(memory_space=pltpu.SMEM),
            pl.BlockSpec((blk, D), lambda i: (i, 0)),
            pl.BlockSpec((MD, D), lambda i: (0, 0)),
        ],
        out_specs=pl.BlockSpec((blk, MD), lambda i: (i, 0)),
        out_shape=jax.ShapeDtypeStruct((B, MD), jnp.float32),
    )


def kernel(ids, embed_weight, proj_weight, scale):
    B = ids.shape[0]
    V, D = embed_weight.shape
    MD = proj_weight.shape[0]
    ids = ids.astype(jnp.int32)
    table3 = embed_weight.reshape(V // 8, 8, D)
    h = _make_gather(V, D, B)(table3, ids)
    mm = _make_matmul(B, D, MD, 2048)
    return mm(scale.reshape(1).astype(jnp.float32), h, proj_weight)
